# baseline (speedup 1.0000x reference)
import functools

import jax
import jax.numpy as jnp
from jax import lax
from jax.experimental import pallas as pl
from jax.experimental.pallas import tpu as pltpu
from jax.experimental.pallas import tpu_sc as plsc

_N = 100000
_C = 512
_NC = 2
_NS = 16
_NW = _NC * _NS
_L = 16
_CHUNK = 3136
_TAIL = _N - (_NW - 1) * _CHUNK
_VECS = _CHUNK // _L
_TVECS = _TAIL // _L
_RED = 4
_COLS = _C // _RED

_mesh = plsc.VectorSubcoreMesh(core_axis_name="c", subcore_axis_name="s",
                               num_cores=_NC, num_subcores=_NS)


@functools.partial(
    pl.kernel,
    out_type=jax.ShapeDtypeStruct((_NC, _C), jnp.float32),
    mesh=_mesh,
    scratch_types=[
        pltpu.VMEM((_CHUNK,), jnp.float32),
        pltpu.VMEM((_CHUNK,), jnp.int32),
        pltpu.VMEM((_C,), jnp.float32),
        pltpu.VMEM((_C,), jnp.float32),
        pltpu.VMEM((_C,), jnp.float32),
        pltpu.VMEM((_C,), jnp.float32),
        pltpu.VMEM((_C,), jnp.float32),
        pltpu.VMEM((_NS, _COLS), jnp.float32),
        pltpu.VMEM((_COLS,), jnp.float32),
        pltpu.VMEM_SHARED((_NS, _C), jnp.float32),
    ],
    compiler_params=pltpu.CompilerParams(needs_layout_passes=False),
)
def _seg_sum_sc(inputs_hbm, labels_hbm, out_hbm,
                vals_v, labs_v, acc_q0, acc_q1, acc_q2, acc_q3,
                acc_v, all_v, out_v, shared):
    cid = lax.axis_index("c")
    sid = lax.axis_index("s")
    wid = sid * _NC + cid
    base = wid * _CHUNK
    is_tail = wid == _NW - 1

    @pl.when(jnp.logical_not(is_tail))
    def _():
        pltpu.sync_copy(inputs_hbm.at[pl.ds(base, _CHUNK)], vals_v)
        pltpu.sync_copy(labels_hbm.at[pl.ds(base, _CHUNK)], labs_v)

    @pl.when(is_tail)
    def _():
        pltpu.sync_copy(inputs_hbm.at[pl.ds(base, _TAIL)],
                        vals_v.at[pl.ds(0, _TAIL)])
        pltpu.sync_copy(labels_hbm.at[pl.ds(base, _TAIL)],
                        labs_v.at[pl.ds(0, _TAIL)])

    accq = [acc_q0, acc_q1, acc_q2, acc_q3]
    zeros = jnp.zeros((_L,), jnp.float32)
    for q in range(4):
        for j in range(_C // _L):
            accq[q][pl.ds(j * _L, _L)] = zeros

    qvecs = _VECS // 4

    def body4(i, carry):
        for q in range(4):
            o = (q * qvecs + i) * _L
            lv = labs_v[pl.ds(o, _L)]
            vv = vals_v[pl.ds(o, _L)]
            plsc.addupdate_scatter(accq[q], [lv], vv)
        return carry

    def body_tail(i, carry):
        lv = labs_v[pl.ds(i * _L, _L)]
        vv = vals_v[pl.ds(i * _L, _L)]
        plsc.addupdate_scatter(acc_q0, [lv], vv)
        return carry

    @pl.when(jnp.logical_not(is_tail))
    def _():
        lax.fori_loop(0, qvecs, body4, 0, unroll=2)

    @pl.when(is_tail)
    def _():
        lax.fori_loop(0, _TVECS, body_tail, 0, unroll=4)

    for j in range(_C // _L):
        sl = pl.ds(j * _L, _L)
        acc_v[sl] = ((acc_q0[sl] + acc_q1[sl])
                     + (acc_q2[sl] + acc_q3[sl]))

    pltpu.sync_copy(acc_v, shared.at[sid])
    plsc.subcore_barrier()

    @pl.when(sid < _RED)
    def _():
        col0 = sid * _COLS
        pltpu.sync_copy(shared.at[:, pl.ds(col0, _COLS)], all_v)
        nacc = _COLS // _L
        accs = [zeros] * nacc
        for r in range(_NS):
            for g in range(nacc):
                accs[g] = accs[g] + all_v[r, pl.ds(g * _L, _L)]
        for g in range(nacc):
            out_v[pl.ds(g * _L, _L)] = accs[g]
        pltpu.sync_copy(out_v, out_hbm.at[cid, pl.ds(col0, _COLS)])


def kernel(inputs, labels):
    partial = _seg_sum_sc(inputs, labels.astype(jnp.int32))
    return partial[0] + partial[1]


# speedup vs baseline: 1.0551x; 1.0551x over previous
import functools

import jax
import jax.numpy as jnp
from jax import lax
from jax.experimental import pallas as pl
from jax.experimental.pallas import tpu as pltpu
from jax.experimental.pallas import tpu_sc as plsc

_N = 100000
_C = 512
_NC = 2
_NS = 16
_NW = _NC * _NS
_L = 16
_CHUNK = 3136
_TAIL = _N - (_NW - 1) * _CHUNK
_VECS = _CHUNK // _L
_TVECS = _TAIL // _L
_RED = 4
_COLS = _C // _RED

_mesh = plsc.VectorSubcoreMesh(core_axis_name="c", subcore_axis_name="s",
                               num_cores=_NC, num_subcores=_NS)


@functools.partial(
    pl.kernel,
    out_type=jax.ShapeDtypeStruct((_NC, _C), jnp.float32),
    mesh=_mesh,
    scratch_types=[
        pltpu.VMEM((_CHUNK,), jnp.float32),
        pltpu.VMEM((_CHUNK,), jnp.int32),
        pltpu.VMEM((_C,), jnp.float32),
        pltpu.VMEM((_NS, _COLS), jnp.float32),
        pltpu.VMEM((_COLS,), jnp.float32),
        pltpu.VMEM_SHARED((_NS, _C), jnp.float32),
    ],
    compiler_params=pltpu.CompilerParams(needs_layout_passes=False),
)
def _seg_sum_sc(inputs_hbm, labels_hbm, out_hbm,
                vals_v, labs_v, acc_v, all_v, out_v, shared):
    cid = lax.axis_index("c")
    sid = lax.axis_index("s")
    wid = sid * _NC + cid
    base = wid * _CHUNK
    is_tail = wid == _NW - 1

    @pl.when(jnp.logical_not(is_tail))
    def _():
        pltpu.sync_copy(inputs_hbm.at[pl.ds(base, _CHUNK)], vals_v)
        pltpu.sync_copy(labels_hbm.at[pl.ds(base, _CHUNK)], labs_v)

    @pl.when(is_tail)
    def _():
        pltpu.sync_copy(inputs_hbm.at[pl.ds(base, _TAIL)],
                        vals_v.at[pl.ds(0, _TAIL)])
        pltpu.sync_copy(labels_hbm.at[pl.ds(base, _TAIL)],
                        labs_v.at[pl.ds(0, _TAIL)])

    zeros = jnp.zeros((_L,), jnp.float32)
    for j in range(_C // _L):
        acc_v[pl.ds(j * _L, _L)] = zeros

    def body(i):
        lv = labs_v[pl.ds(i, _L)]
        vv = vals_v[pl.ds(i, _L)]
        plsc.addupdate_scatter(acc_v, [lv], vv)

    @pl.when(jnp.logical_not(is_tail))
    def _():
        plsc.parallel_loop(0, _CHUNK, _L, unroll=4)(body)

    @pl.when(is_tail)
    def _():
        plsc.parallel_loop(0, _TAIL, _L, unroll=4)(body)

    pltpu.sync_copy(acc_v, shared.at[sid])
    plsc.subcore_barrier()

    @pl.when(sid < _RED)
    def _():
        col0 = sid * _COLS
        pltpu.sync_copy(shared.at[:, pl.ds(col0, _COLS)], all_v)
        nacc = _COLS // _L
        accs = [zeros] * nacc
        for r in range(_NS):
            for g in range(nacc):
                accs[g] = accs[g] + all_v[r, pl.ds(g * _L, _L)]
        for g in range(nacc):
            out_v[pl.ds(g * _L, _L)] = accs[g]
        pltpu.sync_copy(out_v, out_hbm.at[cid, pl.ds(col0, _COLS)])


def kernel(inputs, labels):
    partial = _seg_sum_sc(inputs, labels.astype(jnp.int32))
    return partial[0] + partial[1]
